# baseline (speedup 1.0000x reference)
import functools

import jax
import jax.numpy as jnp
from jax import lax
from jax.experimental import pallas as pl
from jax.experimental.pallas import tpu as pltpu
from jax.experimental.pallas import tpu_sc as plsc

N_EX = 100000
N_CAT = 1000
D = 64
BPAD = 132
SEQ = 200
B = 4096

NW = 32
BT = B // 128
BPW = 128
LANES = 16
NSLOT = 2


@functools.partial(
    pl.kernel,
    mesh=plsc.VectorSubcoreMesh(core_axis_name="c", subcore_axis_name="s"),
    out_type=jax.ShapeDtypeStruct((SEQ, D // 8, BT, 8, 128), jnp.float32),
    compiler_params=pltpu.CompilerParams(use_tc_tiling_on_sc=False,
                                         needs_layout_passes=False),
    scratch_types=(
        [
            pltpu.VMEM((SEQ, BPW), jnp.int32),
            pltpu.VMEM((SEQ, BPW), jnp.int32),
            pltpu.VMEM((SEQ, D), jnp.bfloat16),
        ]
        + [pltpu.VMEM((BPW, D), jnp.bfloat16)
           for _ in range(2 * NSLOT)]
        + [pltpu.VMEM((D // 8, 8, BPAD), jnp.float32)
           for _ in range(NSLOT)]
        + [pltpu.SemaphoreType.DMA for _ in range(3 * NSLOT)]
    ),
)
def _emb_kernel(ex_idx_hbm, cat_idx_hbm, ex_tab, cat_tab, pos_hbm, out_hbm,
                eidx, cidx, pos_v, *bufs):
    exb = bufs[0:2 * NSLOT:2]
    catb = bufs[1:2 * NSLOT:2]
    res = bufs[2 * NSLOT:3 * NSLOT]
    sem_e = bufs[3 * NSLOT:4 * NSLOT]
    sem_c = bufs[4 * NSLOT:5 * NSLOT]
    sem_o = bufs[5 * NSLOT:6 * NSLOT]

    wid = lax.axis_index("s") * 2 + lax.axis_index("c")

    pltpu.sync_copy(ex_idx_hbm.at[wid], eidx)
    pltpu.sync_copy(cat_idx_hbm.at[wid], cidx)
    pltpu.sync_copy(pos_hbm, pos_v)

    iota = jax.lax.iota(jnp.int32, LANES)
    dtv, div = [], []
    for g in range(D // 32):
        for par in range(2):
            dvec = jnp.broadcast_to(32 * g + par, (LANES,)) + iota * 2
            dtv.append(dvec // 8)
            div.append(dvec % 8)

    def gathers(s, k):
        pltpu.async_copy(ex_tab.at[eidx.at[s]], exb[k], sem_e[k])
        pltpu.async_copy(cat_tab.at[cidx.at[s]], catb[k], sem_c[k])

    def wait_gathers(s, k):
        pltpu.make_async_copy(ex_tab.at[eidx.at[s]], exb[k], sem_e[k]).wait()
        pltpu.make_async_copy(cat_tab.at[cidx.at[s]], catb[k],
                              sem_c[k]).wait()

    def out_ref(s):
        return out_hbm.at[s, :, wid]

    def valu(s, k):
        pvec = [pos_v[s, pl.ds(32 * g, 32)] for g in range(D // 32)]

        @plsc.parallel_loop(0, BPW, 1, unroll=8)
        def b_body(b):
            b_vec = jnp.broadcast_to(b, (LANES,))
            for g in range(D // 32):
                sl = pl.ds(32 * g, 32)
                psum = exb[k][b, sl] + catb[k][b, sl] + pvec[g]
                sva, svb = plsc.unpack(psum,
                                       format=plsc.PackFormat.INTERLEAVED)
                plsc.store_scatter(res[k], [dtv[2 * g], div[2 * g], b_vec],
                                   sva)
                plsc.store_scatter(res[k], [dtv[2 * g + 1], div[2 * g + 1],
                                            b_vec], svb)

    for k in range(NSLOT):
        gathers(k, k)

    def loop_body(t, carry):
        a = NSLOT * t
        for k in range(NSLOT):
            s = a + k
            wait_gathers(s, k)

            @pl.when(t > 0)
            def _(k=k, s=s):
                pltpu.make_async_copy(res[k].at[:, :, pl.ds(0, 128)],
                                      out_ref(s - NSLOT), sem_o[k]).wait()

            valu(s, k)
            pltpu.async_copy(res[k].at[:, :, pl.ds(0, 128)], out_ref(s),
                             sem_o[k])

            @pl.when(t < SEQ // NSLOT - 1)
            def _(k=k, s=s):
                gathers(s + NSLOT, k)

        return carry

    lax.fori_loop(0, SEQ // NSLOT, loop_body, 0)

    for k in range(NSLOT):
        pltpu.make_async_copy(res[k].at[:, :, pl.ds(0, 128)],
                              out_ref(SEQ - NSLOT + k), sem_o[k]).wait()


def kernel(exercises, categories, exercise_embed, category_embed,
           position_embed):
    ex_idx = exercises.reshape(NW, BPW, SEQ).transpose(0, 2, 1)
    cat_idx = categories.reshape(NW, BPW, SEQ).transpose(0, 2, 1)
    out5 = _emb_kernel(ex_idx.astype(jnp.int32), cat_idx.astype(jnp.int32),
                       exercise_embed.astype(jnp.bfloat16),
                       category_embed.astype(jnp.bfloat16),
                       position_embed.astype(jnp.bfloat16))
    return out5.transpose(2, 4, 0, 1, 3).reshape(B, SEQ, D)


# speedup vs baseline: 1.0048x; 1.0048x over previous
import functools

import jax
import jax.numpy as jnp
from jax import lax
from jax.experimental import pallas as pl
from jax.experimental.pallas import tpu as pltpu
from jax.experimental.pallas import tpu_sc as plsc

N_EX = 100000
N_CAT = 1000
D = 64
BPAD = 132
SEQ = 200
B = 4096

NW = 32
BT = B // 128
BPW = 128
LANES = 16
NSLOT = 2
NGB = 3
BODY = 6
NT = 198 // BODY


@functools.partial(
    pl.kernel,
    mesh=plsc.VectorSubcoreMesh(core_axis_name="c", subcore_axis_name="s"),
    out_type=jax.ShapeDtypeStruct((SEQ, D // 8, BT, 8, 128), jnp.float32),
    compiler_params=pltpu.CompilerParams(use_tc_tiling_on_sc=False,
                                         needs_layout_passes=False),
    scratch_types=(
        [
            pltpu.VMEM((SEQ, BPW), jnp.int32),
            pltpu.VMEM((SEQ, BPW), jnp.int32),
            pltpu.VMEM((SEQ, D), jnp.bfloat16),
        ]
        + [pltpu.VMEM((BPW, D), jnp.bfloat16)
           for _ in range(2 * NGB)]
        + [pltpu.VMEM((D // 8, 8, BPAD), jnp.float32)
           for _ in range(NSLOT)]
        + [pltpu.SemaphoreType.DMA for _ in range(2 * NGB + NSLOT)]
    ),
)
def _emb_kernel(ex_idx_hbm, cat_idx_hbm, ex_tab, cat_tab, pos_hbm, out_hbm,
                eidx, cidx, pos_v, *bufs):
    exb = bufs[0:2 * NGB:2]
    catb = bufs[1:2 * NGB:2]
    res = bufs[2 * NGB:2 * NGB + NSLOT]
    sems = bufs[2 * NGB + NSLOT:]
    sem_e = sems[0:NGB]
    sem_c = sems[NGB:2 * NGB]
    sem_o = sems[2 * NGB:]

    wid = lax.axis_index("s") * 2 + lax.axis_index("c")

    pltpu.sync_copy(ex_idx_hbm.at[wid], eidx)
    pltpu.sync_copy(cat_idx_hbm.at[wid], cidx)
    pltpu.sync_copy(pos_hbm, pos_v)

    iota = jax.lax.iota(jnp.int32, LANES)
    dtv, div = [], []
    for g in range(D // 32):
        for par in range(2):
            dvec = jnp.broadcast_to(32 * g + par, (LANES,)) + iota * 2
            dtv.append(dvec // 8)
            div.append(dvec % 8)

    def gathers(s, k):
        pltpu.async_copy(ex_tab.at[eidx.at[s]], exb[k], sem_e[k])
        pltpu.async_copy(cat_tab.at[cidx.at[s]], catb[k], sem_c[k])

    def wait_gathers(s, k):
        pltpu.make_async_copy(ex_tab.at[eidx.at[s]], exb[k], sem_e[k]).wait()
        pltpu.make_async_copy(cat_tab.at[cidx.at[s]], catb[k],
                              sem_c[k]).wait()

    def out_ref(s):
        return out_hbm.at[s, :, wid]

    def valu(s, kg, kr):
        pvec = [pos_v[s, pl.ds(32 * g, 32)] for g in range(D // 32)]

        @plsc.parallel_loop(0, BPW, 1, unroll=8)
        def b_body(b):
            b_vec = jnp.broadcast_to(b, (LANES,))
            for g in range(D // 32):
                sl = pl.ds(32 * g, 32)
                psum = exb[kg][b, sl] + catb[kg][b, sl] + pvec[g]
                sva, svb = plsc.unpack(psum,
                                       format=plsc.PackFormat.INTERLEAVED)
                plsc.store_scatter(res[kr], [dtv[2 * g], div[2 * g], b_vec],
                                   sva)
                plsc.store_scatter(res[kr], [dtv[2 * g + 1], div[2 * g + 1],
                                             b_vec], svb)

    def wait_out(s, kr):
        pltpu.make_async_copy(res[kr].at[:, :, pl.ds(0, 128)],
                              out_ref(s), sem_o[kr]).wait()

    gathers(0, 0)
    gathers(1, 1)

    def loop_body(t, carry):
        a = BODY * t
        for j in range(BODY):
            s = a + j
            wait_gathers(s, j % NGB)
            gathers(s + 2, (j + 2) % NGB)
            if j >= 2:
                wait_out(s - 2, j % NSLOT)
            else:

                @pl.when(t > 0)
                def _(s=s, j=j):
                    wait_out(s - 2, j % NSLOT)

            valu(s, j % NGB, j % NSLOT)
            pltpu.async_copy(res[j % NSLOT].at[:, :, pl.ds(0, 128)],
                             out_ref(s), sem_o[j % NSLOT])

        return carry

    lax.fori_loop(0, NT, loop_body, 0)

    for s in (SEQ - 2, SEQ - 1):
        wait_gathers(s, s % NGB)
        wait_out(s - 2, s % NSLOT)
        valu(s, s % NGB, s % NSLOT)
        pltpu.async_copy(res[s % NSLOT].at[:, :, pl.ds(0, 128)], out_ref(s),
                         sem_o[s % NSLOT])
    wait_out(SEQ - 2, 0)
    wait_out(SEQ - 1, 1)


def kernel(exercises, categories, exercise_embed, category_embed,
           position_embed):
    ex_idx = exercises.reshape(NW, BPW, SEQ).transpose(0, 2, 1)
    cat_idx = categories.reshape(NW, BPW, SEQ).transpose(0, 2, 1)
    out5 = _emb_kernel(ex_idx.astype(jnp.int32), cat_idx.astype(jnp.int32),
                       exercise_embed.astype(jnp.bfloat16),
                       category_embed.astype(jnp.bfloat16),
                       position_embed.astype(jnp.bfloat16))
    return out5.transpose(2, 4, 0, 1, 3).reshape(B, SEQ, D)
